# baseline (speedup 1.0000x reference)
import functools

import jax
import jax.numpy as jnp
from jax import lax
from jax.experimental import pallas as pl
from jax.experimental.pallas import tpu as pltpu
from jax.experimental.pallas import tpu_sc as plsc

N_NODES = 10000
N_EDGES = 320000
D = 128

NC = 2
NS = 16
L = 16
NW = NC * NS

C = 64
K = 4
BLK = 16
N_PAD = 10240
CHUNKS = (N_EDGES + C - 1) // C
CHUNKS_PAD = ((CHUNKS + NW * K - 1) // (NW * K)) * (NW * K)
E_PAD = CHUNKS_PAD * C
CH_PER_W = CHUNKS_PAD // NW
CH_PER_CORE = CHUNKS_PAD // NC
NODES_PER_S = N_PAD // NS

_mesh = plsc.VectorSubcoreMesh(
    core_axis_name="c", subcore_axis_name="s", num_cores=NC, num_subcores=NS)

_sc_params = pltpu.CompilerParams(needs_layout_passes=False)


@functools.partial(
    pl.kernel,
    out_type=jax.ShapeDtypeStruct((CHUNKS_PAD, C), jnp.float32),
    mesh=_mesh,
    scratch_types=[
        pltpu.VMEM((N_PAD,), jnp.float32),
        pltpu.VMEM((NS, NODES_PER_S), jnp.float32),
        pltpu.VMEM((BLK, C), jnp.int32),
        pltpu.VMEM((BLK, C), jnp.float32),
        pltpu.VMEM((BLK, C), jnp.float32),
        pltpu.VMEM_SHARED((NS, N_PAD), jnp.float32),
    ],
    compiler_params=_sc_params,
)
def _sc_degw(dst_hbm, ew_hbm, ewp_hbm, deg_loc, red_v, dst_v, ew_v, out_v,
             deg_sh):
  cid = lax.axis_index("c")
  sid = lax.axis_index("s")

  z = jnp.zeros((L,), jnp.float32)

  @pl.loop(0, N_PAD // L)
  def _(i):
    deg_loc[pl.ds(i * L, L)] = z

  @pl.loop(0, CHUNKS_PAD // BLK // NS)
  def _(i):
    b = i * NS + sid
    pltpu.sync_copy(dst_hbm.at[pl.ds(b * BLK, BLK)], dst_v)
    for q in range(BLK):
      for j in range(C // L):
        eid = ((b * BLK + q) * C + j * L
               + lax.broadcasted_iota(jnp.int32, (L,), 0))
        ones = jnp.where(eid < N_EDGES, 1.0, 0.0).astype(jnp.float32)
        idx = dst_v[q, pl.ds(j * L, L)]
        plsc.addupdate_scatter(deg_loc, [idx], ones)

  pltpu.sync_copy(deg_loc, deg_sh.at[sid])
  plsc.subcore_barrier()

  nbase = sid * NODES_PER_S
  for k in range(NS):
    pltpu.sync_copy(deg_sh.at[k, pl.ds(nbase, NODES_PER_S)], red_v.at[k])

  @pl.loop(0, NODES_PER_S // L)
  def _(j):
    acc = red_v[0, pl.ds(j * L, L)]
    for k in range(1, NS):
      acc = acc + red_v[k, pl.ds(j * L, L)]
    deg_loc[pl.ds(nbase + j * L, L)] = acc

  plsc.subcore_barrier()
  pltpu.sync_copy(deg_loc.at[pl.ds(nbase, NODES_PER_S)],
                  deg_sh.at[0, pl.ds(nbase, NODES_PER_S)])
  plsc.subcore_barrier()
  pltpu.sync_copy(deg_sh.at[0], deg_loc)

  @pl.loop(0, CH_PER_W // BLK)
  def _(i):
    ch0 = cid * CH_PER_CORE + sid * CH_PER_W + i * BLK
    pltpu.sync_copy(dst_hbm.at[pl.ds(ch0, BLK)], dst_v)
    pltpu.sync_copy(ew_hbm.at[pl.ds(ch0, BLK)], ew_v)
    for q in range(BLK):
      for j in range(C // L):
        idx = dst_v[q, pl.ds(j * L, L)]
        deg = plsc.load_gather(deg_loc, [idx])
        w = ew_v[q, pl.ds(j * L, L)] / jnp.maximum(deg, 1.0)
        out_v[q, pl.ds(j * L, L)] = w
    pltpu.sync_copy(out_v, ewp_hbm.at[pl.ds(ch0, BLK)])


@functools.partial(
    pl.kernel,
    out_type=jax.ShapeDtypeStruct((NC, N_PAD, D), jnp.float32),
    mesh=_mesh,
    scratch_types=[
        pltpu.VMEM((K, C), jnp.int32),
        pltpu.VMEM((K, C), jnp.int32),
        pltpu.VMEM((K, C), jnp.float32),
        pltpu.VMEM((K, C, D), jnp.float32),
        [pltpu.SemaphoreType.DMA] * K,
        [pltpu.SemaphoreType.DMA] * K,
        [pltpu.SemaphoreType.DMA] * K,
        pltpu.VMEM_SHARED((N_PAD, D), jnp.float32),
    ],
)
def _sc_agg(h_hbm, src_hbm, dst_hbm, ewp_hbm, out_hbm,
            src_v, dst_v, ew_v, rows_v, isem, gsem, ssem, acc):
  cid = lax.axis_index("c")
  sid = lax.axis_index("s")
  wstart = cid * CH_PER_CORE + sid * CH_PER_W
  n = CH_PER_W

  def issue_idx(s, ch):
    pltpu.async_copy(src_hbm.at[wstart + ch], src_v.at[s], isem[s])
    pltpu.async_copy(dst_hbm.at[wstart + ch], dst_v.at[s], isem[s])
    pltpu.async_copy(ewp_hbm.at[wstart + ch], ew_v.at[s], isem[s])

  def wait_idx(s):
    pltpu.make_async_copy(src_hbm.at[0], src_v.at[s], isem[s]).wait()
    pltpu.make_async_copy(dst_hbm.at[0], dst_v.at[s], isem[s]).wait()
    pltpu.make_async_copy(ewp_hbm.at[0], ew_v.at[s], isem[s]).wait()

  def issue_gather(s):
    pltpu.async_copy(h_hbm.at[src_v.at[s]], rows_v.at[s], gsem[s])

  def wait_gather(s):
    pltpu.make_async_copy(h_hbm.at[pl.ds(0, C)], rows_v.at[s], gsem[s]).wait()

  def issue_scatter(s):
    pltpu.async_copy(rows_v.at[s], acc.at[dst_v.at[s]], ssem[s], add=True)

  def wait_scatter(s):
    pltpu.make_async_copy(rows_v.at[s], acc.at[pl.ds(0, C)], ssem[s]).wait()

  def scale(s, ch):
    @plsc.parallel_loop(0, C // L)
    def _(g):
      wg = ew_v[s, pl.ds(g * L, L)]
      for r in range(L):
        row = g * L + r
        w = lax.gather(
            wg,
            jnp.full((L, 1), r, jnp.int32),
            lax.GatherDimensionNumbers(
                offset_dims=(), collapsed_slice_dims=(0,),
                start_index_map=(0,)),
            (1,),
            mode=lax.GatherScatterMode.PROMISE_IN_BOUNDS)
        for v in range(D // L):
          rows_v[s, row, pl.ds(v * L, L)] = (
              rows_v[s, row, pl.ds(v * L, L)] * w)

  z = jnp.zeros((L,), jnp.float32)

  @pl.loop(0, C)
  def _(r):
    for v in range(D // L):
      rows_v[0, r, pl.ds(v * L, L)] = z

  nbase = sid * NODES_PER_S
  for i in range(NODES_PER_S // C):
    pltpu.sync_copy(rows_v.at[0], acc.at[pl.ds(nbase + i * C, C)])

  for q in range(3):
    issue_idx(q, q)
  wait_idx(0)
  issue_gather(0)
  plsc.subcore_barrier()

  @pl.loop(0, n // K)
  def _(g):
    for s in range(K):
      t = g * K + s
      bq = (s + 3) % K

      @pl.when(t + 3 < n)
      def _(t=t, bq=bq):
        @pl.when(t >= 1)
        def _():
          wait_scatter(bq)
        issue_idx(bq, t + 3)

      b1 = (s + 1) % K

      @pl.when(t + 1 < n)
      def _(b1=b1):
        wait_idx(b1)
        issue_gather(b1)

      wait_gather(s)
      scale(s, t)
      issue_scatter(s)

  for s in range(K):
    wait_scatter(s)

  plsc.subcore_barrier()
  for i in range(NODES_PER_S // C):
    pltpu.sync_copy(acc.at[pl.ds(nbase + i * C, C)],
                    out_hbm.at[cid, pl.ds(nbase + i * C, C)])


def _tc_body(relu, h_ref, p_ref, ws_ref, wn_ref, b_ref, o_ref):
  neigh = p_ref[0] + p_ref[1]
  out = (jnp.dot(h_ref[...], ws_ref[...], preferred_element_type=jnp.float32)
         + jnp.dot(neigh, wn_ref[...], preferred_element_type=jnp.float32)
         + b_ref[...])
  if relu:
    out = jnp.maximum(out, 0.0)
  o_ref[...] = out


def _tc_combine(h, partials, ws, wn, b, relu):
  blk = 2048
  return pl.pallas_call(
      functools.partial(_tc_body, relu),
      grid=(N_PAD // blk,),
      in_specs=[
          pl.BlockSpec((blk, D), lambda i: (i, 0)),
          pl.BlockSpec((NC, blk, D), lambda i: (0, i, 0)),
          pl.BlockSpec((D, D), lambda i: (0, 0)),
          pl.BlockSpec((D, D), lambda i: (0, 0)),
          pl.BlockSpec((1, D), lambda i: (0, 0)),
      ],
      out_specs=pl.BlockSpec((blk, D), lambda i: (i, 0)),
      out_shape=jax.ShapeDtypeStruct((N_PAD, D), jnp.float32),
  )(h, partials, ws, wn, b.reshape(1, D))


@jax.jit
def kernel(g_edge_index, in_feat, edge_weights,
           W_self_0, W_neigh_0, b_0,
           W_self_1, W_neigh_1, b_1,
           W_self_2, W_neigh_2, b_2):
  pad_idx = jnp.arange(E_PAD - N_EDGES, dtype=jnp.int32) % N_NODES
  src = jnp.concatenate([g_edge_index[0], pad_idx]).reshape(CHUNKS_PAD, C)
  dst = jnp.concatenate([g_edge_index[1], pad_idx]).reshape(CHUNKS_PAD, C)
  ew = jnp.pad(edge_weights, (0, E_PAD - N_EDGES)).reshape(CHUNKS_PAD, C)

  ewp = _sc_degw(dst, ew)

  h = jnp.pad(in_feat, ((0, N_PAD - N_NODES), (0, 0)))
  params = [(W_self_0, W_neigh_0, b_0),
            (W_self_1, W_neigh_1, b_1),
            (W_self_2, W_neigh_2, b_2)]
  for li, (ws, wn, b) in enumerate(params):
    partials = _sc_agg(h, src, dst, ewp)
    h = _tc_combine(h, partials, ws, wn, b, relu=(li < 2))
  return h[:N_NODES]


# speedup vs baseline: 1.1212x; 1.1212x over previous
import functools

import jax
import jax.numpy as jnp
from jax import lax
from jax.experimental import pallas as pl
from jax.experimental.pallas import tpu as pltpu
from jax.experimental.pallas import tpu_sc as plsc

N_NODES = 10000
N_EDGES = 320000
D = 128

NC = 2
NS = 16
L = 16
NW = NC * NS

C = 64
K = 4
BLK = 16
N_PAD = 10240
CHUNKS = (N_EDGES + C - 1) // C
CHUNKS_PAD = ((CHUNKS + NW * K - 1) // (NW * K)) * (NW * K)
E_PAD = CHUNKS_PAD * C
CH_PER_W = CHUNKS_PAD // NW
CH_PER_CORE = CHUNKS_PAD // NC
NODES_PER_S = N_PAD // NS

_mesh = plsc.VectorSubcoreMesh(
    core_axis_name="c", subcore_axis_name="s", num_cores=NC, num_subcores=NS)

_sc_params = pltpu.CompilerParams(needs_layout_passes=False)


@functools.partial(
    pl.kernel,
    out_type=jax.ShapeDtypeStruct((CHUNKS_PAD, C), jnp.float32),
    mesh=_mesh,
    scratch_types=[
        pltpu.VMEM((N_PAD,), jnp.float32),
        pltpu.VMEM((NS, NODES_PER_S), jnp.float32),
        pltpu.VMEM((BLK, C), jnp.int32),
        pltpu.VMEM((BLK, C), jnp.float32),
        pltpu.VMEM((BLK, C), jnp.float32),
        pltpu.VMEM_SHARED((NS, N_PAD), jnp.float32),
    ],
    compiler_params=_sc_params,
)
def _sc_degw(dst_hbm, ew_hbm, ewp_hbm, deg_loc, red_v, dst_v, ew_v, out_v,
             deg_sh):
  cid = lax.axis_index("c")
  sid = lax.axis_index("s")

  z = jnp.zeros((L,), jnp.float32)

  @pl.loop(0, N_PAD // L)
  def _(i):
    deg_loc[pl.ds(i * L, L)] = z

  @pl.loop(0, CHUNKS_PAD // BLK // NS)
  def _(i):
    b = i * NS + sid
    pltpu.sync_copy(dst_hbm.at[pl.ds(b * BLK, BLK)], dst_v)
    for q in range(BLK):
      for j in range(C // L):
        eid = ((b * BLK + q) * C + j * L
               + lax.broadcasted_iota(jnp.int32, (L,), 0))
        ones = jnp.where(eid < N_EDGES, 1.0, 0.0).astype(jnp.float32)
        idx = dst_v[q, pl.ds(j * L, L)]
        plsc.addupdate_scatter(deg_loc, [idx], ones)

  pltpu.sync_copy(deg_loc, deg_sh.at[sid])
  plsc.subcore_barrier()

  nbase = sid * NODES_PER_S
  for k in range(NS):
    pltpu.sync_copy(deg_sh.at[k, pl.ds(nbase, NODES_PER_S)], red_v.at[k])

  @pl.loop(0, NODES_PER_S // L)
  def _(j):
    acc = red_v[0, pl.ds(j * L, L)]
    for k in range(1, NS):
      acc = acc + red_v[k, pl.ds(j * L, L)]
    deg_loc[pl.ds(nbase + j * L, L)] = acc

  plsc.subcore_barrier()
  pltpu.sync_copy(deg_loc.at[pl.ds(nbase, NODES_PER_S)],
                  deg_sh.at[0, pl.ds(nbase, NODES_PER_S)])
  plsc.subcore_barrier()
  pltpu.sync_copy(deg_sh.at[0], deg_loc)

  @pl.loop(0, CH_PER_W // BLK)
  def _(i):
    ch0 = cid * CH_PER_CORE + sid * CH_PER_W + i * BLK
    pltpu.sync_copy(dst_hbm.at[pl.ds(ch0, BLK)], dst_v)
    pltpu.sync_copy(ew_hbm.at[pl.ds(ch0, BLK)], ew_v)
    for q in range(BLK):
      for j in range(C // L):
        idx = dst_v[q, pl.ds(j * L, L)]
        deg = plsc.load_gather(deg_loc, [idx])
        w = ew_v[q, pl.ds(j * L, L)] / jnp.maximum(deg, 1.0)
        out_v[q, pl.ds(j * L, L)] = w
    pltpu.sync_copy(out_v, ewp_hbm.at[pl.ds(ch0, BLK)])


@functools.partial(
    pl.kernel,
    out_type=jax.ShapeDtypeStruct((NC, N_PAD, D), jnp.float32),
    mesh=_mesh,
    scratch_types=[
        pltpu.VMEM((K, C), jnp.int32),
        pltpu.VMEM((K, C), jnp.int32),
        pltpu.VMEM((K, C), jnp.float32),
        pltpu.VMEM((K, C, D), jnp.float32),
        [pltpu.SemaphoreType.DMA] * K,
        [pltpu.SemaphoreType.DMA] * K,
        [pltpu.SemaphoreType.DMA] * K,
        pltpu.VMEM_SHARED((N_PAD, D), jnp.float32),
    ],
)
def _sc_agg(h_hbm, src_hbm, dst_hbm, ewp_hbm, out_hbm,
            src_v, dst_v, ew_v, rows_v, isem, gsem, ssem, acc):
  cid = lax.axis_index("c")
  sid = lax.axis_index("s")
  wstart = cid * CH_PER_CORE + sid * CH_PER_W
  n = CH_PER_W

  def issue_idx(s, ch):
    pltpu.async_copy(src_hbm.at[wstart + ch], src_v.at[s], isem[s])
    pltpu.async_copy(dst_hbm.at[wstart + ch], dst_v.at[s], isem[s])
    pltpu.async_copy(ewp_hbm.at[wstart + ch], ew_v.at[s], isem[s])

  def wait_idx(s):
    pltpu.make_async_copy(src_hbm.at[0], src_v.at[s], isem[s]).wait()
    pltpu.make_async_copy(dst_hbm.at[0], dst_v.at[s], isem[s]).wait()
    pltpu.make_async_copy(ewp_hbm.at[0], ew_v.at[s], isem[s]).wait()

  def issue_gather(s):
    pltpu.async_copy(h_hbm.at[src_v.at[s]], rows_v.at[s], gsem[s])

  def wait_gather(s):
    pltpu.make_async_copy(h_hbm.at[pl.ds(0, C)], rows_v.at[s], gsem[s]).wait()

  def issue_scatter(s):
    pltpu.async_copy(rows_v.at[s], acc.at[dst_v.at[s]], ssem[s], add=True)

  def wait_scatter(s):
    pltpu.make_async_copy(rows_v.at[s], acc.at[pl.ds(0, C)], ssem[s]).wait()

  def scale(s, ch):
    @pl.loop(0, C // L)
    def _(g):
      wg = ew_v[s, pl.ds(g * L, L)]
      for r in range(L):
        row = g * L + r
        w = lax.gather(
            wg,
            jnp.full((L, 1), r, jnp.int32),
            lax.GatherDimensionNumbers(
                offset_dims=(), collapsed_slice_dims=(0,),
                start_index_map=(0,)),
            (1,),
            mode=lax.GatherScatterMode.PROMISE_IN_BOUNDS)
        for v in range(D // L):
          rows_v[s, row, pl.ds(v * L, L)] = (
              rows_v[s, row, pl.ds(v * L, L)] * w)

  z = jnp.zeros((L,), jnp.float32)

  @pl.loop(0, C)
  def _(r):
    for v in range(D // L):
      rows_v[0, r, pl.ds(v * L, L)] = z

  nbase = sid * NODES_PER_S
  for i in range(NODES_PER_S // C):
    pltpu.sync_copy(rows_v.at[0], acc.at[pl.ds(nbase + i * C, C)])

  for q in range(3):
    issue_idx(q, q)
  wait_idx(0)
  issue_gather(0)
  plsc.subcore_barrier()

  @pl.loop(0, n // K)
  def _(g):
    for s in range(K):
      t = g * K + s
      bq = (s + 3) % K

      @pl.when(t + 3 < n)
      def _(t=t, bq=bq):
        @pl.when(t >= 1)
        def _():
          wait_scatter(bq)
        issue_idx(bq, t + 3)

      b1 = (s + 1) % K

      @pl.when(t + 1 < n)
      def _(b1=b1):
        wait_idx(b1)
        issue_gather(b1)

      wait_gather(s)
      scale(s, t)
      issue_scatter(s)

  for s in range(K):
    wait_scatter(s)

  plsc.subcore_barrier()
  for i in range(NODES_PER_S // C):
    pltpu.sync_copy(acc.at[pl.ds(nbase + i * C, C)],
                    out_hbm.at[cid, pl.ds(nbase + i * C, C)])


def _tc_body(relu, h_ref, p_ref, ws_ref, wn_ref, b_ref, o_ref):
  neigh = p_ref[0] + p_ref[1]
  out = (jnp.dot(h_ref[...], ws_ref[...], preferred_element_type=jnp.float32)
         + jnp.dot(neigh, wn_ref[...], preferred_element_type=jnp.float32)
         + b_ref[...])
  if relu:
    out = jnp.maximum(out, 0.0)
  o_ref[...] = out


def _tc_combine(h, partials, ws, wn, b, relu):
  blk = 2048
  return pl.pallas_call(
      functools.partial(_tc_body, relu),
      grid=(N_PAD // blk,),
      in_specs=[
          pl.BlockSpec((blk, D), lambda i: (i, 0)),
          pl.BlockSpec((NC, blk, D), lambda i: (0, i, 0)),
          pl.BlockSpec((D, D), lambda i: (0, 0)),
          pl.BlockSpec((D, D), lambda i: (0, 0)),
          pl.BlockSpec((1, D), lambda i: (0, 0)),
      ],
      out_specs=pl.BlockSpec((blk, D), lambda i: (i, 0)),
      out_shape=jax.ShapeDtypeStruct((N_PAD, D), jnp.float32),
  )(h, partials, ws, wn, b.reshape(1, D))


@jax.jit
def kernel(g_edge_index, in_feat, edge_weights,
           W_self_0, W_neigh_0, b_0,
           W_self_1, W_neigh_1, b_1,
           W_self_2, W_neigh_2, b_2):
  pad_idx = jnp.arange(E_PAD - N_EDGES, dtype=jnp.int32) % N_NODES
  src = jnp.concatenate([g_edge_index[0], pad_idx]).reshape(CHUNKS_PAD, C)
  dst = jnp.concatenate([g_edge_index[1], pad_idx]).reshape(CHUNKS_PAD, C)
  ew = jnp.pad(edge_weights, (0, E_PAD - N_EDGES)).reshape(CHUNKS_PAD, C)

  ewp = _sc_degw(dst, ew)

  h = jnp.pad(in_feat, ((0, N_PAD - N_NODES), (0, 0)))
  params = [(W_self_0, W_neigh_0, b_0),
            (W_self_1, W_neigh_1, b_1),
            (W_self_2, W_neigh_2, b_2)]
  for li, (ws, wn, b) in enumerate(params):
    partials = _sc_agg(h, src, dst, ewp)
    h = _tc_combine(h, partials, ws, wn, b, relu=(li < 2))
  return h[:N_NODES]
